# baseline (speedup 1.0000x reference)
import jax
import jax.numpy as jnp
from jax.experimental import pallas as pl
from jax.experimental.pallas import tpu as pltpu

CAPACITY = 100000
D_MODEL = 512
BATCH = 4096

TAIL = CAPACITY - BATCH
N_CHUNKS = 8
CHUNK = 12000
LAST = TAIL - (N_CHUNKS - 1) * CHUNK

IMP_COLS = 128
IMP_ROWS = -(-CAPACITY // IMP_COLS)
IMP_ROWS_PAD = -(-IMP_ROWS // 8) * 8
IMP_PAD = IMP_ROWS_PAD * IMP_COLS
SIG_ROWS = BATCH // IMP_COLS


def _body(emb, sig, mem, imp, out_mem, out_imp, sem_emb, sem_sig, sem_imp,
          sem_tail):
    copies = [
        pltpu.make_async_copy(emb, out_mem.at[pl.ds(0, BATCH)], sem_emb),
        pltpu.make_async_copy(sig, out_imp.at[pl.ds(0, SIG_ROWS)], sem_sig),
        pltpu.make_async_copy(imp.at[pl.ds(SIG_ROWS, IMP_ROWS_PAD - SIG_ROWS)],
                              out_imp.at[pl.ds(SIG_ROWS, IMP_ROWS_PAD - SIG_ROWS)],
                              sem_imp),
    ]
    for c in range(N_CHUNKS):
        start = BATCH + c * CHUNK
        size = LAST if c == N_CHUNKS - 1 else CHUNK
        copies.append(pltpu.make_async_copy(
            mem.at[pl.ds(start, size)], out_mem.at[pl.ds(start, size)],
            sem_tail.at[c]))
    for c in copies:
        c.start()
    for c in copies:
        c.wait()


def kernel(embeddings, loss_signal, memory, importance):
    sig2d = loss_signal.reshape(SIG_ROWS, IMP_COLS)
    imp2d = jnp.pad(importance, (0, IMP_PAD - CAPACITY)).reshape(
        IMP_ROWS_PAD, IMP_COLS)
    any_spec = pl.BlockSpec(memory_space=pl.ANY)
    out_mem, out_imp2d = pl.pallas_call(
        _body,
        in_specs=[any_spec] * 4,
        out_specs=[any_spec] * 2,
        out_shape=[
            jax.ShapeDtypeStruct((CAPACITY, D_MODEL), jnp.float32),
            jax.ShapeDtypeStruct((IMP_ROWS_PAD, IMP_COLS), jnp.float32),
        ],
        scratch_shapes=[
            pltpu.SemaphoreType.DMA,
            pltpu.SemaphoreType.DMA,
            pltpu.SemaphoreType.DMA,
            pltpu.SemaphoreType.DMA((N_CHUNKS,)),
        ],
    )(embeddings, sig2d, memory, imp2d)
    out_imp = out_imp2d.reshape(IMP_PAD)[:CAPACITY]
    return out_mem, out_imp


# speedup vs baseline: 47.0977x; 47.0977x over previous
import jax
import jax.numpy as jnp
from jax.experimental import pallas as pl
from jax.experimental.pallas import tpu as pltpu

CAPACITY = 100000
D_MODEL = 512
BATCH = 4096

BLOCK_ROWS = 4096
NB_EMB = BATCH // BLOCK_ROWS
GRID = (CAPACITY + BLOCK_ROWS - 1) // BLOCK_ROWS

IMP_PAD = GRID * BLOCK_ROWS
IMP_COLS = 128
IMP_ROWS = IMP_PAD // IMP_COLS
IMP_BLOCK_ROWS = BLOCK_ROWS // IMP_COLS
SIG_ROWS = BATCH // IMP_COLS


def _body(emb_ref, sig_ref, mem_ref, imp_ref, out_mem_ref, out_imp_ref):
    i = pl.program_id(0)

    @pl.when(i < NB_EMB)
    def _():
        out_mem_ref[...] = emb_ref[...]
        out_imp_ref[...] = sig_ref[...]

    @pl.when(i >= NB_EMB)
    def _():
        out_mem_ref[...] = mem_ref[...]
        out_imp_ref[...] = imp_ref[...]


def kernel(embeddings, loss_signal, memory, importance):
    sig2d = loss_signal.reshape(SIG_ROWS, IMP_COLS)
    imp2d = jnp.pad(importance, (0, IMP_PAD - CAPACITY)).reshape(IMP_ROWS, IMP_COLS)

    emb_last = NB_EMB - 1
    out_mem, out_imp2d = pl.pallas_call(
        _body,
        grid=(GRID,),
        in_specs=[
            pl.BlockSpec((BLOCK_ROWS, D_MODEL), lambda i: (jnp.minimum(i, emb_last), 0)),
            pl.BlockSpec((IMP_BLOCK_ROWS, IMP_COLS), lambda i: (jnp.minimum(i, emb_last), 0)),
            pl.BlockSpec((BLOCK_ROWS, D_MODEL), lambda i: (jnp.maximum(i, NB_EMB), 0)),
            pl.BlockSpec((IMP_BLOCK_ROWS, IMP_COLS), lambda i: (jnp.maximum(i, NB_EMB), 0)),
        ],
        out_specs=[
            pl.BlockSpec((BLOCK_ROWS, D_MODEL), lambda i: (i, 0)),
            pl.BlockSpec((IMP_BLOCK_ROWS, IMP_COLS), lambda i: (i, 0)),
        ],
        out_shape=[
            jax.ShapeDtypeStruct((CAPACITY, D_MODEL), jnp.float32),
            jax.ShapeDtypeStruct((IMP_ROWS, IMP_COLS), jnp.float32),
        ],
    )(embeddings, sig2d, memory, imp2d)

    out_imp = out_imp2d.reshape(IMP_PAD)[:CAPACITY]
    return out_mem, out_imp
